# baseline (speedup 1.0000x reference)
import jax
import jax.numpy as jnp
from jax.experimental import pallas as pl
from jax.experimental.pallas import tpu as pltpu


def _round_up(x, m):
    return ((x + m - 1) // m) * m


def _cdiv(a, b):
    return (a + b - 1) // b


def _make_kernel(tm, ns, nxbuf, nobuf):
    def _mod_linear_kernel(x_hbm, theta_ref, gamma_ref, bias_ref, out_hbm,
                           x_buf, out_buf, xsem, osem):
        c = pl.program_id(0)
        base = c * (ns * tm)

        g_bf = gamma_ref[...].astype(jnp.bfloat16)
        th = theta_ref[...]
        bs = bias_ref[...]

        def xcopy(s):
            slot = s % nxbuf
            return pltpu.make_async_copy(
                x_hbm.at[pl.ds(base + s * tm, tm), :],
                x_buf.at[slot], xsem.at[slot])

        def ocopy(s):
            slot = s % nobuf
            return pltpu.make_async_copy(
                out_buf.at[slot],
                out_hbm.at[pl.ds(base + s * tm, tm), :], osem.at[slot])

        for s in range(min(nxbuf, ns)):
            xcopy(s).start()
        for s in range(ns):
            xcopy(s).wait()
            if s >= nobuf:
                ocopy(s - nobuf).wait()
            xs = (x_buf[s % nxbuf] * th).astype(jnp.bfloat16)
            acc = jax.lax.dot_general(xs, g_bf, (((1,), (1,)), ((), ())),
                                      preferred_element_type=jnp.float32)
            out_buf[s % nobuf] = (acc + bs).astype(out_buf.dtype)
            ocopy(s).start()
            if s + nxbuf < ns:
                xcopy(s + nxbuf).start()
        for s in range(max(0, ns - nobuf), ns):
            ocopy(s).wait()

    return _mod_linear_kernel


def kernel(x, theta, gamma, bias):
    B, F = x.shape
    T, F2 = gamma.shape
    assert F == F2 and theta.shape == (F,) and bias.shape == (T,)
    dtype = x.dtype

    F_pad = _round_up(F, 128)
    T_pad = _round_up(T, 128)

    tm = min(1024, _round_up(B, 8))
    nc = 2 if B > tm else 1
    ns = _cdiv(B, tm * nc)
    nxbuf = min(3, ns)
    nobuf = min(2, ns)
    B_pad = nc * ns * tm

    x_p = jnp.pad(x, ((0, B_pad - B), (0, F_pad - F)))
    gamma_p = jnp.pad(gamma, ((0, T_pad - T), (0, F_pad - F)))
    theta_p = jnp.pad(theta, (0, F_pad - F)).reshape(1, F_pad)
    bias_p = jnp.pad(bias, (0, T_pad - T)).reshape(1, T_pad)

    out = pl.pallas_call(
        _make_kernel(tm, ns, nxbuf, nobuf),
        out_shape=jax.ShapeDtypeStruct((B_pad, T_pad), dtype),
        grid=(nc,),
        in_specs=[
            pl.BlockSpec(memory_space=pltpu.MemorySpace.HBM),
            pl.BlockSpec((1, F_pad), lambda c: (0, 0)),
            pl.BlockSpec((T_pad, F_pad), lambda c: (0, 0)),
            pl.BlockSpec((1, T_pad), lambda c: (0, 0)),
        ],
        out_specs=pl.BlockSpec(memory_space=pltpu.MemorySpace.HBM),
        scratch_shapes=[
            pltpu.VMEM((nxbuf, tm, F_pad), jnp.float32),
            pltpu.VMEM((nobuf, tm, T_pad), jnp.float32),
            pltpu.SemaphoreType.DMA((nxbuf,)),
            pltpu.SemaphoreType.DMA((nobuf,)),
        ],
        compiler_params=pltpu.CompilerParams(
            dimension_semantics=("parallel",),
            vmem_limit_bytes=56 * 1024 * 1024,
        ),
    )(x_p, theta_p, gamma_p, bias_p)

    return out[:B, :T]


# speedup vs baseline: 1.1275x; 1.1275x over previous
import jax
import jax.numpy as jnp
from jax.experimental import pallas as pl
from jax.experimental.pallas import tpu as pltpu


def _round_up(x, m):
    return ((x + m - 1) // m) * m


def _cdiv(a, b):
    return (a + b - 1) // b


def _make_kernel(tm, r):
    def _mod_linear_kernel(x_ref, theta_ref, gamma_ref, bias_ref, out_ref):
        xs = (x_ref[...] * theta_ref[...]).astype(jnp.bfloat16)
        g_bf = gamma_ref[...].astype(jnp.bfloat16)
        acc = jax.lax.dot_general(xs, g_bf, (((1,), (1,)), ((), ())),
                                  preferred_element_type=jnp.float32)
        res = (acc + bias_ref[...]).astype(out_ref.dtype)
        if r == 1:
            out_ref[...] = res
        else:
            step = pl.program_id(0) * pl.num_programs(1) + pl.program_id(1)
            out_ref[pl.ds((step % r) * tm, tm), :] = res
    return _mod_linear_kernel


def kernel(x, theta, gamma, bias):
    B, F = x.shape
    T, F2 = gamma.shape
    assert F == F2 and theta.shape == (F,) and bias.shape == (T,)
    dtype = x.dtype

    F_pad = _round_up(F, 128)
    T_pad = _round_up(T, 128)

    tm = min(1024, _round_up(B, 8))
    nc = 2 if B > tm else 1
    ns = _cdiv(B, tm * nc)
    r = 1
    B_pad = nc * ns * tm

    x_p = jnp.pad(x, ((0, B_pad - B), (0, F_pad - F)))
    gamma_p = jnp.pad(gamma, ((0, T_pad - T), (0, F_pad - F)))
    theta_p = jnp.pad(theta, (0, F_pad - F)).reshape(1, F_pad)
    bias_p = jnp.pad(bias, (0, T_pad - T)).reshape(1, T_pad)

    out = pl.pallas_call(
        _make_kernel(tm, r),
        out_shape=jax.ShapeDtypeStruct((B_pad, T_pad), dtype),
        grid=(nc, ns),
        in_specs=[
            pl.BlockSpec((tm, F_pad), lambda c, s: (c * ns + s, 0)),
            pl.BlockSpec((1, F_pad), lambda c, s: (0, 0)),
            pl.BlockSpec((T_pad, F_pad), lambda c, s: (0, 0)),
            pl.BlockSpec((1, T_pad), lambda c, s: (0, 0)),
        ],
        out_specs=pl.BlockSpec((r * tm, T_pad),
                               lambda c, s: ((c * ns + s) // r, 0)),
        compiler_params=pltpu.CompilerParams(
            dimension_semantics=("parallel", "arbitrary"),
            vmem_limit_bytes=60 * 1024 * 1024,
        ),
    )(x_p, theta_p, gamma_p, bias_p)

    return out[:B, :T]
